# baseline (speedup 1.0000x reference)
import functools

import jax
import jax.numpy as jnp
from jax import lax
from jax.experimental import pallas as pl
from jax.experimental.pallas import tpu as pltpu
from jax.experimental.pallas import tpu_sc as plsc

NC, NS = 2, 16
NW = NC * NS
B = 4096
H = 50
HP = 64
D = 64
DP = 128
NT2 = 500000
BW = B // NW
L = 16
ND = D // L

_mesh = plsc.VectorSubcoreMesh(
    core_axis_name="c", subcore_axis_name="s", num_cores=NC, num_subcores=NS
)


def _slot(j):
    return j if j < 48 else j + 14


@functools.partial(
    pl.kernel,
    out_type=[
        jax.ShapeDtypeStruct((B,), jnp.float32),
        jax.ShapeDtypeStruct((B,), jnp.float32),
    ],
    mesh=_mesh,
    compiler_params=pltpu.CompilerParams(use_tc_tiling_on_sc=False),
    scratch_types=[
        pltpu.VMEM((BW, H), jnp.int32),
        pltpu.VMEM((BW, H), jnp.float32),
        pltpu.VMEM((BW, HP), jnp.int32),
        pltpu.VMEM((BW,), jnp.int32),
        pltpu.VMEM((BW,), jnp.int32),
        pltpu.VMEM((BW,), jnp.int32),
        pltpu.VMEM((BW,), jnp.int32),
        pltpu.VMEM((BW,), jnp.int32),
        pltpu.VMEM((BW,), jnp.int32),
        pltpu.VMEM((BW, DP), jnp.float32),
        pltpu.VMEM((BW, DP), jnp.float32),
        pltpu.VMEM((BW, DP), jnp.float32),
        pltpu.VMEM((HP, DP), jnp.float32),
        pltpu.VMEM((HP, DP), jnp.float32),
        pltpu.VMEM((BW,), jnp.float32),
        pltpu.VMEM((BW,), jnp.float32),
        pltpu.SemaphoreType.DMA,
        pltpu.SemaphoreType.DMA,
        pltpu.SemaphoreType.DMA,
    ],
)
def _two_tower(uid_hbm, hid_hbm, mask_hbm, pid_hbm, nid_hbm, utab, itab,
               pos_out, neg_out,
               hid_v, mask_v, h2_v, uid_v, pid_v, nid_v, u2_v, p2_v, n2_v,
               urows, prows, nrows, hbuf0, hbuf1, pout_v, nout_v,
               sem_aux, sem_h0, sem_h1):
    wid = lax.axis_index("s") * NC + lax.axis_index("c")
    base = wid * BW

    pltpu.sync_copy(hid_hbm.at[pl.ds(base, BW)], hid_v)
    pltpu.sync_copy(mask_hbm.at[pl.ds(base, BW)], mask_v)
    pltpu.sync_copy(uid_hbm.at[pl.ds(base, BW)], uid_v)
    pltpu.sync_copy(pid_hbm.at[pl.ds(base, BW)], pid_v)
    pltpu.sync_copy(nid_hbm.at[pl.ds(base, BW)], nid_v)

    def paux(i, _):
        u2_v[pl.ds(i * L, L)] = uid_v[pl.ds(i * L, L)] >> 1
        p2_v[pl.ds(i * L, L)] = pid_v[pl.ds(i * L, L)] >> 1
        n2_v[pl.ds(i * L, L)] = nid_v[pl.ds(i * L, L)] >> 1
        return 0

    lax.fori_loop(0, BW // L, paux, 0)

    def phist(b, _):
        for k in range(3):
            h2_v[b, pl.ds(k * L, L)] = hid_v[b, pl.ds(k * L, L)] >> 1
        h2_v[b, pl.ds(48, L)] = hid_v[b, pl.ds(H - L, L)] >> 1
        return 0

    lax.fori_loop(0, BW, phist, 0)

    cu = pltpu.make_async_copy(utab.at[u2_v], urows, sem_aux)
    cp = pltpu.make_async_copy(itab.at[p2_v], prows, sem_aux)
    cn = pltpu.make_async_copy(itab.at[n2_v], nrows, sem_aux)
    cu.start()
    cp.start()
    cn.start()

    pltpu.make_async_copy(itab.at[h2_v.at[0]], hbuf0, sem_h0).start()

    cu.wait()
    cp.wait()
    cn.wait()

    hbufs = (hbuf0, hbuf1)
    sems = (sem_h0, sem_h1)

    lane_iota = lax.iota(jnp.int32, L)

    def lane_sum(v):
        for k in (8, 4, 2, 1):
            v = v + v.at[lane_iota ^ k].get(mode="promise_in_bounds")
        return v

    def bcast(vec, lane):
        return vec.at[jnp.full((L,), lane, jnp.int32)].get(
            mode="promise_in_bounds"
        )

    def compute_row(b, hbuf):
        zero = jnp.zeros((L,), jnp.float32)
        mvecs = [mask_v[b, pl.ds(k * L, L)] for k in range(H // L)]
        mvecs.append(mask_v[b, pl.ds(H - L, L)])
        pvecs = [
            (hid_v[b, pl.ds(k * L, L)] & 1).astype(jnp.float32)
            for k in range(H // L)
        ]
        pvecs.append((hid_v[b, pl.ds(H - L, L)] & 1).astype(jnp.float32))
        accs = [zero] * ND
        hlv = zero
        for j in range(H):
            if j < (H // L) * L:
                k, lane = j // L, j % L
            else:
                k, lane = 3, j - (H - L)
            mv = bcast(mvecs[k], lane)
            pf = bcast(pvecs[k], lane)
            s = _slot(j)
            for d in range(ND):
                h0 = hbuf[s, pl.ds(d * L, L)]
                h1 = hbuf[s, pl.ds(D + d * L, L)]
                accs[d] = accs[d] + (h0 + pf * (h1 - h0)) * mv
            hlv = hlv + mv
        inv = 1.0 / jnp.maximum(hlv, 1.0)

        g = (b // L) * L
        pu = bcast((uid_v[pl.ds(g, L)] & 1).astype(jnp.float32), b - g)
        pp = bcast((pid_v[pl.ds(g, L)] & 1).astype(jnp.float32), b - g)
        pn = bcast((nid_v[pl.ds(g, L)] & 1).astype(jnp.float32), b - g)
        ps = zero
        ns = zero
        for d in range(ND):
            u0 = urows[b, pl.ds(d * L, L)]
            u1 = urows[b, pl.ds(D + d * L, L)]
            p0 = prows[b, pl.ds(d * L, L)]
            p1 = prows[b, pl.ds(D + d * L, L)]
            n0 = nrows[b, pl.ds(d * L, L)]
            n1 = nrows[b, pl.ds(D + d * L, L)]
            uv = (u0 + pu * (u1 - u0)) + accs[d] * inv
            ps = ps + uv * (p0 + pp * (p1 - p0))
            ns = ns + uv * (n0 + pn * (n1 - n0))
        return lane_sum(ps), lane_sum(ns)

    def body(i, carry):
        psv, nsv = carry
        for t in range(2):
            b = 2 * i + t
            nxt = b + 1

            @pl.when(nxt < BW)
            def _start():
                pltpu.make_async_copy(
                    itab.at[h2_v.at[nxt]], hbufs[1 - t], sems[1 - t]
                ).start()

            pltpu.make_async_copy(
                itab.at[h2_v.at[b]], hbufs[t], sems[t]
            ).wait()
            sp, sn = compute_row(b, hbufs[t])
            sel = lane_iota == lax.rem(b, L)
            psv = jnp.where(sel, sp, psv)
            nsv = jnp.where(sel, sn, nsv)

        @pl.when(lax.rem(i, L // 2) == (L // 2 - 1))
        def _store():
            s = 2 * i + 1 - (L - 1)
            pout_v[pl.ds(s, L)] = psv
            nout_v[pl.ds(s, L)] = nsv

        return (psv, nsv)

    zero = jnp.zeros((L,), jnp.float32)
    lax.fori_loop(0, BW // 2, body, (zero, zero))

    pltpu.sync_copy(pout_v, pos_out.at[pl.ds(base, BW)])
    pltpu.sync_copy(nout_v, neg_out.at[pl.ds(base, BW)])


def kernel(user_ids, history_item_ids, history_mask, pos_item_ids,
           neg_item_ids, user_emb_table, item_emb_table):
    pos, neg = _two_tower(
        user_ids.astype(jnp.int32),
        history_item_ids.astype(jnp.int32),
        history_mask,
        pos_item_ids.astype(jnp.int32),
        neg_item_ids.astype(jnp.int32),
        user_emb_table.reshape(NT2, DP),
        item_emb_table.reshape(NT2, DP),
    )
    return pos, neg


# speedup vs baseline: 1.0303x; 1.0303x over previous
import functools

import jax
import jax.numpy as jnp
from jax import lax
from jax.experimental import pallas as pl
from jax.experimental.pallas import tpu as pltpu
from jax.experimental.pallas import tpu_sc as plsc

NC, NS = 2, 16
NW = NC * NS
B = 4096
H = 50
D = 64
BW = B // NW
L = 16
ND = D // L

_mesh = plsc.VectorSubcoreMesh(
    core_axis_name="c", subcore_axis_name="s", num_cores=NC, num_subcores=NS
)


@functools.partial(
    pl.kernel,
    out_type=[
        jax.ShapeDtypeStruct((B,), jnp.float32),
        jax.ShapeDtypeStruct((B,), jnp.float32),
    ],
    mesh=_mesh,
    compiler_params=pltpu.CompilerParams(use_tc_tiling_on_sc=False),
    scratch_types=[
        pltpu.VMEM((BW, H), jnp.int32),
        pltpu.VMEM((BW, H), jnp.float32),
        pltpu.VMEM((BW,), jnp.int32),
        pltpu.VMEM((BW,), jnp.int32),
        pltpu.VMEM((BW,), jnp.int32),
        pltpu.VMEM((BW, D), jnp.float32),
        pltpu.VMEM((BW, D), jnp.float32),
        pltpu.VMEM((BW, D), jnp.float32),
        pltpu.VMEM((H, D), jnp.float32),
        pltpu.VMEM((H, D), jnp.float32),
        pltpu.VMEM((BW,), jnp.float32),
        pltpu.VMEM((BW,), jnp.float32),
        pltpu.SemaphoreType.DMA,
        pltpu.SemaphoreType.DMA,
        pltpu.SemaphoreType.DMA,
    ],
)
def _two_tower(uid_hbm, hid_hbm, mask_hbm, pid_hbm, nid_hbm, utab, itab,
               pos_out, neg_out,
               hid_v, mask_v, uid_v, pid_v, nid_v, urows, prows, nrows,
               hbuf0, hbuf1, pout_v, nout_v, sem_aux, sem_h0, sem_h1):
    wid = lax.axis_index("s") * NC + lax.axis_index("c")
    base = wid * BW

    pltpu.sync_copy(hid_hbm.at[pl.ds(base, BW)], hid_v)
    pltpu.sync_copy(mask_hbm.at[pl.ds(base, BW)], mask_v)
    pltpu.sync_copy(uid_hbm.at[pl.ds(base, BW)], uid_v)
    pltpu.sync_copy(pid_hbm.at[pl.ds(base, BW)], pid_v)
    pltpu.sync_copy(nid_hbm.at[pl.ds(base, BW)], nid_v)

    cu = pltpu.make_async_copy(utab.at[uid_v], urows, sem_aux)
    cp = pltpu.make_async_copy(itab.at[pid_v], prows, sem_aux)
    cn = pltpu.make_async_copy(itab.at[nid_v], nrows, sem_aux)
    cu.start()
    cp.start()
    cn.start()

    pltpu.make_async_copy(itab.at[hid_v.at[0]], hbuf0, sem_h0).start()

    cu.wait()
    cp.wait()
    cn.wait()

    hbufs = (hbuf0, hbuf1)
    sems = (sem_h0, sem_h1)

    lane_iota = lax.iota(jnp.int32, L)

    def lane_sum(v):
        for k in (8, 4, 2, 1):
            v = v + v.at[lane_iota ^ k].get(mode="promise_in_bounds")
        return v

    def compute_row(b, hbuf):
        zero = jnp.zeros((L,), jnp.float32)
        mvecs = [mask_v[b, pl.ds(k * L, L)] for k in range(H // L)]
        mtail = mask_v[b, pl.ds(H - L, L)]
        accs = [zero] * ND
        hlv = zero
        for j in range(H):
            if j < (H // L) * L:
                src, lane = mvecs[j // L], j % L
            else:
                src, lane = mtail, j - (H - L)
            mv = src.at[jnp.full((L,), lane, jnp.int32)].get(
                mode="promise_in_bounds"
            )
            for d in range(ND):
                accs[d] = accs[d] + hbuf[j, pl.ds(d * L, L)] * mv
            hlv = hlv + mv
        inv = 1.0 / jnp.maximum(hlv, 1.0)
        ps = zero
        ns = zero
        for d in range(ND):
            uv = urows[b, pl.ds(d * L, L)] + accs[d] * inv
            ps = ps + uv * prows[b, pl.ds(d * L, L)]
            ns = ns + uv * nrows[b, pl.ds(d * L, L)]
        return lane_sum(ps), lane_sum(ns)

    def body(i, carry):
        psv, nsv = carry
        for t in range(2):
            b = 2 * i + t
            nxt = b + 1

            @pl.when(nxt < BW)
            def _start():
                pltpu.make_async_copy(
                    itab.at[hid_v.at[nxt]], hbufs[1 - t], sems[1 - t]
                ).start()

            pltpu.make_async_copy(
                itab.at[hid_v.at[b]], hbufs[t], sems[t]
            ).wait()
            sp, sn = compute_row(b, hbufs[t])
            sel = lane_iota == lax.rem(b, L)
            psv = jnp.where(sel, sp, psv)
            nsv = jnp.where(sel, sn, nsv)

        @pl.when(lax.rem(i, L // 2) == (L // 2 - 1))
        def _store():
            s = 2 * i + 1 - (L - 1)
            pout_v[pl.ds(s, L)] = psv
            nout_v[pl.ds(s, L)] = nsv

        return (psv, nsv)

    zero = jnp.zeros((L,), jnp.float32)
    lax.fori_loop(0, BW // 2, body, (zero, zero))

    pltpu.sync_copy(pout_v, pos_out.at[pl.ds(base, BW)])
    pltpu.sync_copy(nout_v, neg_out.at[pl.ds(base, BW)])


def kernel(user_ids, history_item_ids, history_mask, pos_item_ids,
           neg_item_ids, user_emb_table, item_emb_table):
    pos, neg = _two_tower(
        user_ids.astype(jnp.int32),
        history_item_ids.astype(jnp.int32),
        history_mask,
        pos_item_ids.astype(jnp.int32),
        neg_item_ids.astype(jnp.int32),
        user_emb_table,
        item_emb_table,
    )
    return pos, neg
